# baseline (speedup 1.0000x reference)
import functools
import math

import jax
import jax.numpy as jnp
from jax import lax
from jax.experimental import pallas as pl
from jax.experimental.pallas import tpu as pltpu
from jax.experimental.pallas import tpu_sc as plsc

DECAY = 0.99
TOP_K = 8


def _proj_kernel(mem_ref, wk_ref, imp_ref, age_ref, kp_ref, bias_ref):
    kp_ref[...] = jnp.dot(mem_ref[...], wk_ref[...],
                          preferred_element_type=jnp.float32)
    eff = imp_ref[...] * jnp.exp(age_ref[...] * math.log(DECAY))
    bias_ref[...] = jnp.maximum(jnp.log(eff), -10.0)


def _select_kernel(q_ref, wq_ref, kp_ref, bias_ref, mem_ref,
                   idx_ref, w_ref, ret_ref, s_ref):
    tl = q_ref.shape[1]
    d = q_ref.shape[-1]
    qp = jnp.dot(q_ref[0], wq_ref[...], preferred_element_type=jnp.float32)
    s = jax.lax.dot_general(qp, kp_ref[...], (((1,), (1,)), ((), ())),
                            preferred_element_type=jnp.float32)
    s = s * (1.0 / math.sqrt(d)) + bias_ref[...]
    s_ref[...] = s

    n_slots = s.shape[-1]
    neg_inf = jnp.float32(-jnp.inf)
    iota = jax.lax.broadcasted_iota(jnp.int32, (tl, n_slots), 1)

    work = s
    ms, idxs = [], []
    for i in range(TOP_K):
        m = jnp.max(work, axis=1, keepdims=True)
        eq = work == m
        idxs.append(jnp.max(jnp.where(eq, iota, -1), axis=1, keepdims=True))
        ms.append(m)
        work = jnp.where(eq, neg_inf, work)
    m0 = ms[0]
    sel = work == neg_inf
    e = jnp.where(sel, jnp.exp(s - m0), 0.0)
    denom = jnp.sum(e, axis=1, keepdims=True)
    attn = e / denom
    ret_ref[0] = jnp.dot(attn, mem_ref[...],
                         preferred_element_type=jnp.float32)
    idx_ref[...] = jnp.concatenate(idxs, axis=1)
    w_ref[...] = jnp.concatenate(
        [jnp.exp(m - m0) / denom for m in ms], axis=1)
    n_sel = jnp.sum(sel.astype(jnp.float32))

    @pl.when(n_sel != float(TOP_K * tl))
    def _exact_repair():
        sc = s_ref[...]
        work2 = sc
        idxs2, ms2 = [], []
        for _ in range(TOP_K):
            m = jnp.max(work2, axis=1, keepdims=True)
            first = jnp.min(jnp.where(work2 == m, iota, n_slots), axis=1,
                            keepdims=True)
            idxs2.append(first)
            ms2.append(m)
            work2 = jnp.where(iota == first, neg_inf, work2)
        mr = ms2[0]
        e2 = jnp.where(work2 == neg_inf, jnp.exp(sc - mr), 0.0)
        den2 = jnp.sum(e2, axis=1, keepdims=True)
        ret_ref[0] = jnp.dot(e2 / den2, mem_ref[...],
                             preferred_element_type=jnp.float32)
        idx_ref[...] = jnp.concatenate(idxs2, axis=1)
        w_ref[...] = jnp.concatenate(
            [jnp.exp(m - mr) / den2 for m in ms2], axis=1)


def _make_sc_scatter(R, S, nw):
    rpw = R // nw
    nchunks = rpw // 16
    mesh = plsc.VectorSubcoreMesh(core_axis_name="c", subcore_axis_name="s")

    del nchunks
    nblk = rpw // 128

    @functools.partial(
        pl.kernel, mesh=mesh,
        out_type=jax.ShapeDtypeStruct((R * S,), jnp.float32),
        scratch_types=[
            pltpu.VMEM((TOP_K, rpw), jnp.int32),
            pltpu.VMEM((TOP_K, rpw), jnp.float32),
            pltpu.VMEM((16 * S,), jnp.float32),
            pltpu.VMEM((TOP_K * nblk, 128), jnp.int32),
            pltpu.SemaphoreType.DMA,
        ],
    )
    def sc_scatter(idx_hbm, w_hbm, zeros_hbm, out_hbm, idx_v, w_v, ztile_v,
                   pos_v, sem):
        nc = jax.lax.axis_size("c")
        wid = lax.axis_index("s") * nc + lax.axis_index("c")
        base = wid * rpw
        iota16 = lax.iota(jnp.int32, 16)
        pltpu.sync_copy(idx_hbm.at[wid], idx_v)
        pltpu.sync_copy(w_hbm.at[wid], w_v)
        pltpu.sync_copy(zeros_hbm, ztile_v)

        def zfill(c, carry):
            pltpu.sync_copy(
                ztile_v, out_hbm.at[pl.ds((base + c * 16) * S, 16 * S)])
            return carry

        lax.fori_loop(0, rpw // 16, zfill, 0)

        def posbuild(jj, carry):
            j = jj // nblk
            r0 = (jj % nblk) * 128
            for k in range(8):
                rows = base + r0 + k * 16 + iota16
                cols = idx_v[j, pl.ds(r0 + k * 16, 16)]
                pos_v[jj, pl.ds(k * 16, 16)] = rows * S + cols
            return carry

        lax.fori_loop(0, TOP_K * nblk, posbuild, 0)

        def scatter(jj, carry):
            j = jj // nblk
            r0 = (jj % nblk) * 128
            pltpu.async_copy(w_v.at[j, pl.ds(r0, 128)],
                             out_hbm.at[pos_v.at[jj]], sem).wait()
            return carry

        lax.fori_loop(0, TOP_K * nblk, scatter, 0)

    return sc_scatter


def kernel(query, memory, importance, age, W_q, W_k, top_k):
    B, L, d = query.shape
    S = memory.shape[1]
    R = B * L
    mem2d = memory.reshape(S, d)

    kp, bias = pl.pallas_call(
        _proj_kernel,
        out_shape=[
            jax.ShapeDtypeStruct((S, d), jnp.float32),
            jax.ShapeDtypeStruct((1, S), jnp.float32),
        ],
    )(mem2d, W_k, importance, age)

    tl = min(512, L)
    nlt = L // tl
    grid = (B, nlt)
    idx, w, ret = pl.pallas_call(
        _select_kernel,
        grid=grid,
        in_specs=[
            pl.BlockSpec((1, tl, d), lambda b, l: (b, l, 0)),
            pl.BlockSpec((d, d), lambda b, l: (0, 0)),
            pl.BlockSpec((S, d), lambda b, l: (0, 0)),
            pl.BlockSpec((1, S), lambda b, l: (0, 0)),
            pl.BlockSpec((S, d), lambda b, l: (0, 0)),
        ],
        out_specs=[
            pl.BlockSpec((tl, TOP_K), lambda b, l: (b * nlt + l, 0)),
            pl.BlockSpec((tl, TOP_K), lambda b, l: (b * nlt + l, 0)),
            pl.BlockSpec((1, tl, d), lambda b, l: (b, l, 0)),
        ],
        out_shape=[
            jax.ShapeDtypeStruct((R, TOP_K), jnp.int32),
            jax.ShapeDtypeStruct((R, TOP_K), jnp.float32),
            jax.ShapeDtypeStruct((B, L, d), jnp.float32),
        ],
        scratch_shapes=[
            pltpu.VMEM((tl, S), jnp.float32),
        ],
        compiler_params=pltpu.CompilerParams(
            dimension_semantics=("parallel", "parallel")),
    )(query, W_q, kp, bias, mem2d)

    info = plsc.get_sparse_core_info()
    nw = info.num_cores * info.num_subcores
    rpw = R // nw
    idx_t = idx.reshape(nw, rpw, TOP_K).transpose(0, 2, 1)
    w_t = w.reshape(nw, rpw, TOP_K).transpose(0, 2, 1)
    zeros = jnp.zeros((16 * S,), jnp.float32)
    attn_flat = _make_sc_scatter(R, S, nw)(idx_t, w_t, zeros)
    return ret, attn_flat.reshape(B, L, S)


# speedup vs baseline: 3.8464x; 3.8464x over previous
import math

import jax
import jax.numpy as jnp
from jax.experimental import pallas as pl
from jax.experimental.pallas import tpu as pltpu

DECAY = 0.99
TOP_K = 8


def _proj_kernel(mem_ref, wk_ref, imp_ref, age_ref, kp_ref, bias_ref):
    kp_ref[...] = jnp.dot(mem_ref[...], wk_ref[...],
                          preferred_element_type=jnp.float32)
    eff = imp_ref[...] * jnp.exp(age_ref[...] * math.log(DECAY))
    bias_ref[...] = jnp.maximum(jnp.log(eff), -10.0)


def _attn_kernel(q_ref, wq_ref, kp_ref, bias_ref, mem_ref, attn_ref, ret_ref,
                 s_ref):
    tl = q_ref.shape[1]
    d = q_ref.shape[-1]
    qp = jnp.dot(q_ref[0], wq_ref[...], preferred_element_type=jnp.float32)
    s = jax.lax.dot_general(qp, kp_ref[...], (((1,), (1,)), ((), ())),
                            preferred_element_type=jnp.float32)
    s = s * (1.0 / math.sqrt(d)) + bias_ref[...]
    s_ref[...] = s

    n_slots = s.shape[-1]
    neg_inf = jnp.float32(-jnp.inf)

    work = s
    m0 = None
    for i in range(TOP_K):
        m = jnp.max(work, axis=1, keepdims=True)
        if i == 0:
            m0 = m
        work = jnp.where(work == m, neg_inf, work)
    sel = work == neg_inf
    e = jnp.where(sel, jnp.exp(s - m0), 0.0)
    attn_ref[0] = e / jnp.sum(e, axis=1, keepdims=True)
    n_sel = jnp.sum(sel.astype(jnp.float32))

    @pl.when(n_sel != float(TOP_K * tl))
    def _exact_repair():
        iota = jax.lax.broadcasted_iota(jnp.int32, (tl, n_slots), 1)
        sc = s_ref[...]
        work2 = sc
        for _ in range(TOP_K):
            m = jnp.max(work2, axis=1, keepdims=True)
            first = jnp.min(jnp.where(work2 == m, iota, n_slots), axis=1,
                            keepdims=True)
            work2 = jnp.where(iota == first, neg_inf, work2)
        mr = jnp.max(sc, axis=1, keepdims=True)
        e2 = jnp.where(work2 == neg_inf, jnp.exp(sc - mr), 0.0)
        attn_ref[0] = e2 / jnp.sum(e2, axis=1, keepdims=True)

    ret_ref[0] = jnp.dot(attn_ref[0], mem_ref[...],
                         preferred_element_type=jnp.float32)


def kernel(query, memory, importance, age, W_q, W_k, top_k):
    B, L, d = query.shape
    S = memory.shape[1]
    mem2d = memory.reshape(S, d)

    kp, bias = pl.pallas_call(
        _proj_kernel,
        out_shape=[
            jax.ShapeDtypeStruct((S, d), jnp.float32),
            jax.ShapeDtypeStruct((1, S), jnp.float32),
        ],
    )(mem2d, W_k, importance, age)

    tl = min(2048, L)
    grid = (B, L // tl)
    attn, ret = pl.pallas_call(
        _attn_kernel,
        grid=grid,
        in_specs=[
            pl.BlockSpec((1, tl, d), lambda b, l: (b, l, 0)),
            pl.BlockSpec((d, d), lambda b, l: (0, 0)),
            pl.BlockSpec((S, d), lambda b, l: (0, 0)),
            pl.BlockSpec((1, S), lambda b, l: (0, 0)),
            pl.BlockSpec((S, d), lambda b, l: (0, 0)),
        ],
        out_specs=[
            pl.BlockSpec((1, tl, S), lambda b, l: (b, l, 0)),
            pl.BlockSpec((1, tl, d), lambda b, l: (b, l, 0)),
        ],
        out_shape=[
            jax.ShapeDtypeStruct((B, L, S), jnp.float32),
            jax.ShapeDtypeStruct((B, L, d), jnp.float32),
        ],
        scratch_shapes=[
            pltpu.VMEM((tl, S), jnp.float32),
        ],
        compiler_params=pltpu.CompilerParams(
            dimension_semantics=("parallel", "parallel")),
    )(query, W_q, kp, bias, mem2d)
    return ret, attn


# speedup vs baseline: 3.9329x; 1.0225x over previous
import math

import jax
import jax.numpy as jnp
from jax.experimental import pallas as pl
from jax.experimental.pallas import tpu as pltpu

DECAY = 0.99
TOP_K = 8


def _attn_kernel(q_ref, wq_ref, wk_ref, imp_ref, age_ref, mem_ref,
                 attn_ref, ret_ref, s_ref, w_ref, kp_ref, bias_ref):
    tl = q_ref.shape[1]
    d = q_ref.shape[-1]

    @pl.when(pl.program_id(0) + pl.program_id(1) == 0)
    def _proj():
        kp_ref[...] = jnp.dot(mem_ref[...], wk_ref[...],
                              preferred_element_type=jnp.float32)
        eff = imp_ref[...] * jnp.exp(age_ref[...] * math.log(DECAY))
        bias_ref[...] = jnp.maximum(jnp.log(eff), -10.0)

    qp = jnp.dot(q_ref[0], wq_ref[...], preferred_element_type=jnp.float32)
    s = jax.lax.dot_general(qp, kp_ref[...], (((1,), (1,)), ((), ())),
                            preferred_element_type=jnp.float32)
    s = s * (1.0 / math.sqrt(d)) + bias_ref[...]
    s_ref[...] = s

    n_slots = s.shape[-1]
    neg_inf = jnp.float32(-jnp.inf)

    work = s
    m0 = None
    for i in range(TOP_K):
        m = jnp.max(work, axis=1, keepdims=True)
        if i == 0:
            m0 = m
        work = jnp.where(work == m, neg_inf, work)
    w_ref[...] = work
    n_sel = jnp.sum((work == neg_inf).astype(jnp.float32))

    @pl.when(n_sel != float(TOP_K * tl))
    def _exact_repair():
        iota = jax.lax.broadcasted_iota(jnp.int32, (tl, n_slots), 1)
        work2 = s_ref[...]
        for _ in range(TOP_K):
            m = jnp.max(work2, axis=1, keepdims=True)
            first = jnp.min(jnp.where(work2 == m, iota, n_slots), axis=1,
                            keepdims=True)
            work2 = jnp.where(iota == first, neg_inf, work2)
        w_ref[...] = work2

    sel = w_ref[...] == neg_inf
    e = jnp.where(sel, jnp.exp(s_ref[...] - m0), 0.0)
    attn = e / jnp.sum(e, axis=1, keepdims=True)
    attn_ref[0] = attn
    ret_ref[0] = jnp.dot(attn, mem_ref[...],
                         preferred_element_type=jnp.float32)


def kernel(query, memory, importance, age, W_q, W_k, top_k):
    B, L, d = query.shape
    S = memory.shape[1]
    mem2d = memory.reshape(S, d)
    tl = min(2048, L)
    grid = (B, L // tl)
    attn, ret = pl.pallas_call(
        _attn_kernel,
        grid=grid,
        in_specs=[
            pl.BlockSpec((1, tl, d), lambda b, l: (b, l, 0)),
            pl.BlockSpec((d, d), lambda b, l: (0, 0)),
            pl.BlockSpec((d, d), lambda b, l: (0, 0)),
            pl.BlockSpec((1, S), lambda b, l: (0, 0)),
            pl.BlockSpec((1, S), lambda b, l: (0, 0)),
            pl.BlockSpec((S, d), lambda b, l: (0, 0)),
        ],
        out_specs=[
            pl.BlockSpec((1, tl, S), lambda b, l: (b, l, 0)),
            pl.BlockSpec((1, tl, d), lambda b, l: (b, l, 0)),
        ],
        out_shape=[
            jax.ShapeDtypeStruct((B, L, S), jnp.float32),
            jax.ShapeDtypeStruct((B, L, d), jnp.float32),
        ],
        scratch_shapes=[
            pltpu.VMEM((tl, S), jnp.float32),
            pltpu.VMEM((tl, S), jnp.float32),
            pltpu.VMEM((S, d), jnp.float32),
            pltpu.VMEM((1, S), jnp.float32),
        ],
        compiler_params=pltpu.CompilerParams(
            dimension_semantics=("arbitrary", "arbitrary")),
    )(query, W_q, W_k, importance, age, mem2d)
    return ret, attn
